# baseline (speedup 1.0000x reference)
import jax
import jax.numpy as jnp
from jax.experimental import pallas as pl
from jax.experimental.pallas import tpu as pltpu


def _make_kernel(nc, bt, bcast_rows, n_copies):
    def _moe_kernel(x_hbm, gw_ref, gb_ref, ew_hbm, eb_ref, out_ref,
                    xbuf, bcast, w_vmem, w_sem, x_sem, o_sem):
        w_cp = pltpu.make_async_copy(ew_hbm, w_vmem, w_sem)
        w_cp.start()
        x_cps = [
            pltpu.make_async_copy(
                x_hbm.at[pl.ds(j * bt, bt), :], xbuf.at[j], x_sem.at[j])
            for j in range(nc)
        ]
        for cp in x_cps:
            cp.start()

        s = jnp.zeros((gw_ref.shape[0], gw_ref.shape[1]), jnp.float32)
        c = jnp.zeros((1, gw_ref.shape[0]), jnp.float32)
        for j in range(nc):
            x_cps[j].wait()
            xb = xbuf[j]
            logits = jax.lax.dot_general(
                xb, gw_ref[...], dimension_numbers=(((1,), (1,)), ((), ())),
                preferred_element_type=jnp.float32)
            logits = logits + gb_ref[...]
            m = jnp.max(logits, axis=-1, keepdims=True)
            p = jnp.exp(logits - m)
            w = p / jnp.sum(p, axis=-1, keepdims=True)
            s = s + jax.lax.dot_general(
                w, xb, dimension_numbers=(((0,), (0,)), ((), ())),
                preferred_element_type=jnp.float32)
            c = c + jnp.sum(w, axis=0, keepdims=True)

        w_cp.wait()
        acc = jax.lax.dot_general(
            c, eb_ref[...], dimension_numbers=(((1,), (0,)), ((), ())),
            preferred_element_type=jnp.float32)
        for e in range(w_vmem.shape[0]):
            acc = acc + jax.lax.dot_general(
                s[e:e + 1, :], w_vmem[e],
                dimension_numbers=(((1,), (1,)), ((), ())),
                preferred_element_type=jnp.float32)
        bcast[...] = jnp.broadcast_to(acc, bcast.shape)
        o_cps = [
            pltpu.make_async_copy(
                bcast, out_ref.at[pl.ds(j * bcast_rows, bcast_rows), :],
                o_sem)
            for j in range(n_copies)
        ]
        for cp in o_cps:
            cp.start()
        for cp in o_cps:
            cp.wait()

    return _moe_kernel


def kernel(x, expert_w, expert_b, gate_w, gate_b):
    B, I = x.shape
    E, O, _ = expert_w.shape
    BT = 1024
    BCAST_ROWS = 512
    nc = B // BT
    n_copies = B // BCAST_ROWS
    out = pl.pallas_call(
        _make_kernel(nc, BT, BCAST_ROWS, n_copies),
        in_specs=[
            pl.BlockSpec(memory_space=pl.ANY),
            pl.BlockSpec(memory_space=pltpu.MemorySpace.VMEM),
            pl.BlockSpec(memory_space=pltpu.MemorySpace.VMEM),
            pl.BlockSpec(memory_space=pl.ANY),
            pl.BlockSpec(memory_space=pltpu.MemorySpace.VMEM),
        ],
        out_specs=pl.BlockSpec(memory_space=pl.ANY),
        out_shape=jax.ShapeDtypeStruct((B, O), jnp.float32),
        scratch_shapes=[pltpu.VMEM((nc, BT, I), jnp.float32),
                        pltpu.VMEM((BCAST_ROWS, O), jnp.float32),
                        pltpu.VMEM((E, O, I), jnp.float32),
                        pltpu.SemaphoreType.DMA,
                        pltpu.SemaphoreType.DMA((B // BT,)),
                        pltpu.SemaphoreType.DMA],
    )(x, gate_w, gate_b.reshape(1, E), expert_w, expert_b)
    return out


# speedup vs baseline: 1.2753x; 1.2753x over previous
import jax
import jax.numpy as jnp
from jax.experimental import pallas as pl
from jax.experimental.pallas import tpu as pltpu


def _make_kernel(nb, bcast_rows, n_copies):
    def _moe_kernel(x_ref, gw_ref, gb_ref, ew_ref, eb_ref, out_ref,
                    s_acc, c_acc, bcast, w_vmem, w_sem, o_sem):
        k = pl.program_id(0)

        @pl.when(k == 0)
        def _init():
            for e in range(w_vmem.shape[0]):
                pltpu.make_async_copy(
                    ew_ref.at[e], w_vmem.at[e], w_sem.at[e]).start()
            s_acc[...] = jnp.zeros_like(s_acc)
            c_acc[...] = jnp.zeros_like(c_acc)

        xb = x_ref[...]
        logits = jax.lax.dot_general(
            xb, gw_ref[...], dimension_numbers=(((1,), (1,)), ((), ())),
            preferred_element_type=jnp.float32)
        logits = logits + gb_ref[...]
        m = jnp.max(logits, axis=-1, keepdims=True)
        p = jnp.exp(logits - m)
        w = p / jnp.sum(p, axis=-1, keepdims=True)
        s_acc[...] += jax.lax.dot_general(
            w, xb, dimension_numbers=(((0,), (0,)), ((), ())),
            preferred_element_type=jnp.float32)
        c_acc[...] += jnp.sum(w, axis=0, keepdims=True)

        @pl.when(k == nb - 1)
        def _finish():
            s = s_acc[...]
            acc = jax.lax.dot_general(
                c_acc[...], eb_ref[...],
                dimension_numbers=(((1,), (0,)), ((), ())),
                preferred_element_type=jnp.float32)
            for e in range(w_vmem.shape[0]):
                pltpu.make_async_copy(
                    ew_ref.at[e], w_vmem.at[e], w_sem.at[e]).wait()
                acc = acc + jax.lax.dot_general(
                    s[e:e + 1, :], w_vmem[e],
                    dimension_numbers=(((1,), (1,)), ((), ())),
                    preferred_element_type=jnp.float32)
            bcast[...] = jnp.broadcast_to(acc, bcast.shape)
            copies = [
                pltpu.make_async_copy(
                    bcast, out_ref.at[pl.ds(j * bcast_rows, bcast_rows), :],
                    o_sem)
                for j in range(n_copies)
            ]
            for cp in copies:
                cp.start()
            for cp in copies:
                cp.wait()

    return _moe_kernel


def kernel(x, expert_w, expert_b, gate_w, gate_b):
    B, I = x.shape
    E, O, _ = expert_w.shape
    BT = 2048
    BCAST_ROWS = 512
    nb = B // BT
    n_copies = B // BCAST_ROWS
    out = pl.pallas_call(
        _make_kernel(nb, BCAST_ROWS, n_copies),
        grid=(nb,),
        in_specs=[
            pl.BlockSpec((BT, I), lambda k: (k, 0)),
            pl.BlockSpec((E, I), lambda k: (0, 0)),
            pl.BlockSpec((1, E), lambda k: (0, 0)),
            pl.BlockSpec(memory_space=pl.ANY),
            pl.BlockSpec((E, O), lambda k: (0, 0)),
        ],
        out_specs=pl.BlockSpec(memory_space=pl.ANY),
        out_shape=jax.ShapeDtypeStruct((B, O), jnp.float32),
        scratch_shapes=[pltpu.VMEM((E, I), jnp.float32),
                        pltpu.VMEM((1, E), jnp.float32),
                        pltpu.VMEM((BCAST_ROWS, O), jnp.float32),
                        pltpu.VMEM((E, O, I), jnp.float32),
                        pltpu.SemaphoreType.DMA((E,)),
                        pltpu.SemaphoreType.DMA],
    )(x, gate_w, gate_b.reshape(1, E), expert_w, expert_b)
    return out
